# baseline (speedup 1.0000x reference)
import jax
import jax.numpy as jnp
from jax.experimental import pallas as pl
from jax.experimental.pallas import tpu as pltpu

N, D, H = 10000, 128, 128
BM = 200


def _p1_body(adj_ref, xm_ref, x_ref, w1t_ref, b1_ref, eps_ref, h_ref):
    s = jnp.dot(adj_ref[...], x_ref[...], preferred_element_type=jnp.float32)
    out = s + (1.0 + eps_ref[0]) * xm_ref[...]
    h_ref[...] = (
        jnp.dot(out, w1t_ref[...], preferred_element_type=jnp.float32)
        + b1_ref[...]
    )


def _p2_body(h_ref, g_ref, bt_ref, w2t_ref, b2_ref, o_ref):
    h = h_ref[...]
    mu = jnp.mean(h, axis=0, keepdims=True)
    var = jnp.mean((h - mu) ** 2, axis=0, keepdims=True)
    hn = (h - mu) / jnp.sqrt(var + 1e-5) * g_ref[...] + bt_ref[...]
    hn = jnp.maximum(hn, 0.0)
    o_ref[...] = (
        jnp.dot(hn, w2t_ref[...], preferred_element_type=jnp.float32)
        + b2_ref[...]
    )


def kernel(x, adj, eps, W1, b1, gamma1, beta1, W2, b2):
    h = pl.pallas_call(
        _p1_body,
        grid=(N // BM,),
        in_specs=[
            pl.BlockSpec((BM, N), lambda i: (i, 0)),
            pl.BlockSpec((BM, D), lambda i: (i, 0)),
            pl.BlockSpec((N, D), lambda i: (0, 0)),
            pl.BlockSpec((D, H), lambda i: (0, 0)),
            pl.BlockSpec((1, H), lambda i: (0, 0)),
            pl.BlockSpec(memory_space=pltpu.SMEM),
        ],
        out_specs=pl.BlockSpec((BM, H), lambda i: (i, 0)),
        out_shape=jax.ShapeDtypeStruct((N, H), jnp.float32),
        compiler_params=pltpu.CompilerParams(
            dimension_semantics=("parallel",)
        ),
    )(adj, x, x, W1.T, b1.reshape(1, H), eps)

    out = pl.pallas_call(
        _p2_body,
        in_specs=[
            pl.BlockSpec((N, H), lambda: (0, 0)),
            pl.BlockSpec((1, H), lambda: (0, 0)),
            pl.BlockSpec((1, H), lambda: (0, 0)),
            pl.BlockSpec((H, D), lambda: (0, 0)),
            pl.BlockSpec((1, D), lambda: (0, 0)),
        ],
        out_specs=pl.BlockSpec((N, D), lambda: (0, 0)),
        out_shape=jax.ShapeDtypeStruct((N, D), jnp.float32),
    )(h, gamma1.reshape(1, H), beta1.reshape(1, H), W2.T,
      b2.reshape(1, D))
    return out


# speedup vs baseline: 1.0172x; 1.0172x over previous
import jax
import jax.numpy as jnp
from jax.experimental import pallas as pl
from jax.experimental.pallas import tpu as pltpu

N, D, H = 10000, 128, 128
BM = 400


def _p1_body(adj_ref, xm_ref, x_ref, w1t_ref, b1_ref, eps_ref, h_ref):
    s = jnp.dot(adj_ref[...], x_ref[...], preferred_element_type=jnp.float32)
    out = s + (1.0 + eps_ref[0]) * xm_ref[...]
    h_ref[...] = (
        jnp.dot(out, w1t_ref[...], preferred_element_type=jnp.float32)
        + b1_ref[...]
    )


def _p2_body(h_ref, g_ref, bt_ref, w2t_ref, b2_ref, o_ref):
    h = h_ref[...]
    mu = jnp.mean(h, axis=0, keepdims=True)
    var = jnp.mean((h - mu) ** 2, axis=0, keepdims=True)
    hn = (h - mu) / jnp.sqrt(var + 1e-5) * g_ref[...] + bt_ref[...]
    hn = jnp.maximum(hn, 0.0)
    o_ref[...] = (
        jnp.dot(hn, w2t_ref[...], preferred_element_type=jnp.float32)
        + b2_ref[...]
    )


def kernel(x, adj, eps, W1, b1, gamma1, beta1, W2, b2):
    h = pl.pallas_call(
        _p1_body,
        grid=(N // BM,),
        in_specs=[
            pl.BlockSpec((BM, N), lambda i: (i, 0)),
            pl.BlockSpec((BM, D), lambda i: (i, 0)),
            pl.BlockSpec((N, D), lambda i: (0, 0)),
            pl.BlockSpec((D, H), lambda i: (0, 0)),
            pl.BlockSpec((1, H), lambda i: (0, 0)),
            pl.BlockSpec(memory_space=pltpu.SMEM),
        ],
        out_specs=pl.BlockSpec((BM, H), lambda i: (i, 0)),
        out_shape=jax.ShapeDtypeStruct((N, H), jnp.float32),
        compiler_params=pltpu.CompilerParams(
            dimension_semantics=("parallel",)
        ),
    )(adj, x, x, W1.T, b1.reshape(1, H), eps)

    out = pl.pallas_call(
        _p2_body,
        in_specs=[
            pl.BlockSpec((N, H), lambda: (0, 0)),
            pl.BlockSpec((1, H), lambda: (0, 0)),
            pl.BlockSpec((1, H), lambda: (0, 0)),
            pl.BlockSpec((H, D), lambda: (0, 0)),
            pl.BlockSpec((1, D), lambda: (0, 0)),
        ],
        out_specs=pl.BlockSpec((N, D), lambda: (0, 0)),
        out_shape=jax.ShapeDtypeStruct((N, D), jnp.float32),
    )(h, gamma1.reshape(1, H), beta1.reshape(1, H), W2.T,
      b2.reshape(1, D))
    return out


# speedup vs baseline: 1.0948x; 1.0762x over previous
import jax
import jax.numpy as jnp
from jax.experimental import pallas as pl
from jax.experimental.pallas import tpu as pltpu

N, D, H = 10000, 128, 128
BM = 400
NB = N // BM

_DN = (((1,), (1,)), ((), ()))


def _body(adj_ref, x_ref, w1_ref, b1_ref, g_ref, bt_ref, w2_ref, b2_ref,
          eps_ref, out_ref, sum_ref):
    i = pl.program_id(0)

    @pl.when(i < NB)
    def _stream():
        s = jnp.dot(adj_ref[...], x_ref[...],
                    preferred_element_type=jnp.float32)
        xm = x_ref[pl.ds(i * BM, BM), :]
        agg = s + (1.0 + eps_ref[0]) * xm
        h = jax.lax.dot_general(agg, w1_ref[...], _DN,
                                preferred_element_type=jnp.float32)
        h = h + b1_ref[...]
        out_ref[pl.ds(i * BM, BM), :] = h
        colsum = jnp.sum(h, axis=0, keepdims=True)

        @pl.when(i == 0)
        def _():
            sum_ref[...] = colsum

        @pl.when(i > 0)
        def _():
            sum_ref[...] += colsum

    @pl.when(i == NB)
    def _finalize():
        h = out_ref[...]
        mu = sum_ref[...] * (1.0 / N)
        var = jnp.mean((h - mu) ** 2, axis=0, keepdims=True)
        hn = (h - mu) / jnp.sqrt(var + 1e-5) * g_ref[...] + bt_ref[...]
        hn = jnp.maximum(hn, 0.0)
        o = jax.lax.dot_general(hn, w2_ref[...], _DN,
                                preferred_element_type=jnp.float32)
        out_ref[...] = o + b2_ref[...]


def kernel(x, adj, eps, W1, b1, gamma1, beta1, W2, b2):
    return pl.pallas_call(
        _body,
        grid=(NB + 1,),
        in_specs=[
            pl.BlockSpec((BM, N), lambda i: (jnp.minimum(i, NB - 1), 0)),
            pl.BlockSpec((N, D), lambda i: (0, 0)),
            pl.BlockSpec((H, D), lambda i: (0, 0)),
            pl.BlockSpec((1, H), lambda i: (0, 0)),
            pl.BlockSpec((1, H), lambda i: (0, 0)),
            pl.BlockSpec((1, H), lambda i: (0, 0)),
            pl.BlockSpec((D, H), lambda i: (0, 0)),
            pl.BlockSpec((1, D), lambda i: (0, 0)),
            pl.BlockSpec(memory_space=pltpu.SMEM),
        ],
        out_specs=pl.BlockSpec((N, D), lambda i: (0, 0)),
        out_shape=jax.ShapeDtypeStruct((N, D), jnp.float32),
        scratch_shapes=[pltpu.VMEM((1, H), jnp.float32)],
        compiler_params=pltpu.CompilerParams(
            dimension_semantics=("arbitrary",)
        ),
    )(adj, x, W1, b1.reshape(1, H), gamma1.reshape(1, H),
      beta1.reshape(1, H), W2, b2.reshape(1, D), eps)
